# Initial kernel scaffold; baseline (speedup 1.0000x reference)
import jax
import jax.numpy as jnp
from jax.experimental import pallas as pl


def kernel(input1, input2):
    raise NotImplementedError("write your pallas kernel here")


# speedup vs baseline: 1.5819x; 1.5819x over previous
import jax
import jax.numpy as jnp
from jax.experimental import pallas as pl

_B, _N, _M, _D = 8, 2048, 2048, 3
_TN = 512


def _chamfer_block(x_ref, y_ref, d1_ref, d2_ref):
    n = pl.program_id(1)
    xb = x_ref[0]
    yb = y_ref[0]
    d = None
    for k in range(_D):
        dk = xb[k][:, None] - yb[k][None, :]
        d = dk * dk if d is None else d + dk * dk
    d1_ref[0, 0, :] = jnp.min(d, axis=1)
    pmin = jnp.min(d, axis=0)

    @pl.when(n == 0)
    def _init():
        d2_ref[0, 0, :] = pmin

    @pl.when(n != 0)
    def _acc():
        d2_ref[0, 0, :] = jnp.minimum(d2_ref[0, 0, :], pmin)


@jax.jit
def kernel(input1, input2):
    x = jnp.transpose(input1, (0, 2, 1))
    y = jnp.transpose(input2, (0, 2, 1))
    d1, d2 = pl.pallas_call(
        _chamfer_block,
        grid=(_B, _N // _TN),
        in_specs=[
            pl.BlockSpec((1, _D, _TN), lambda b, n: (b, 0, n)),
            pl.BlockSpec((1, _D, _M), lambda b, n: (b, 0, 0)),
        ],
        out_specs=[
            pl.BlockSpec((1, 1, _TN), lambda b, n: (b, 0, n)),
            pl.BlockSpec((1, 1, _M), lambda b, n: (b, 0, 0)),
        ],
        out_shape=[
            jax.ShapeDtypeStruct((_B, 1, _N), jnp.float32),
            jax.ShapeDtypeStruct((_B, 1, _M), jnp.float32),
        ],
    )(x, y)
    return (d1[:, 0, :], d2[:, 0, :])
